# Initial kernel scaffold; baseline (speedup 1.0000x reference)
import jax
import jax.numpy as jnp
from jax.experimental import pallas as pl

_TOP_K = 4
_EPS = 1e-06


def _obm_kernel(k_ref, v_ref, q_ref, o_ref):
    k = k_ref[0, 0]
    v = v_ref[0, 0]
    q = q_ref[0, 0]
    S, D = k.shape
    iota = jax.lax.broadcasted_iota(jnp.int32, (S, D), 1)

    ak = jnp.abs(k)
    kmax = jnp.max(ak, axis=1, keepdims=True)
    a_idx = jnp.min(jnp.where(ak == kmax, iota, D), axis=1, keepdims=True)
    pt = (iota == a_idx).astype(jnp.float32)

    dn = (((1,), (1,)), ((), ()))
    dc = (((1,), (0,)), ((), ()))
    A = jax.lax.dot_general(q, k, dn, preferred_element_type=jnp.float32)
    asum = jax.lax.dot_general(A, pt, dc, preferred_element_type=jnp.float32)

    work = jnp.abs(q)
    scores = []
    onehots = []
    for _ in range(_TOP_K):
        m = jnp.max(work, axis=1, keepdims=True)
        idx = jnp.min(jnp.where(work == m, iota, D), axis=1, keepdims=True)
        oh = iota == idx
        scores.append(m)
        onehots.append(oh)
        work = jnp.where(oh, -jnp.inf, work)
    es = [jnp.exp(s - scores[0]) for s in scores]
    w_norm = es[0]
    for e in es[1:]:
        w_norm = w_norm + e

    g = jnp.zeros((S, D), jnp.float32)
    for e, oh in zip(es, onehots):
        seg = jnp.sum(jnp.where(oh, asum, 0.0), axis=1, keepdims=True) + _EPS
        g = g + jnp.where(oh, (e / w_norm) / seg, 0.0)

    G = jax.lax.dot_general(g, pt, dn, preferred_element_type=jnp.float32)
    C = A * G
    o_ref[0, 0] = jax.lax.dot_general(C, v, dc, preferred_element_type=jnp.float32)


@jax.jit
def kernel(keys, values, queries):
    Bb, H, S, D = keys.shape
    spec = pl.BlockSpec((1, 1, S, D), lambda b, h: (b, h, 0, 0))
    return pl.pallas_call(
        _obm_kernel,
        grid=(Bb, H),
        in_specs=[spec, spec, spec],
        out_specs=spec,
        out_shape=jax.ShapeDtypeStruct((Bb, H, S, D), jnp.float32),
    )(keys, values, queries)


# speedup vs baseline: 18.9902x; 18.9902x over previous
import jax
import jax.numpy as jnp
from jax.experimental import pallas as pl

_TOP_K = 4
_EPS = 1e-06


def _obm_kernel(k_ref, v_ref, q_ref, o_ref):
    k = k_ref[0, 0]
    v = v_ref[0, 0]
    q = q_ref[0, 0]
    S, D = k.shape
    iota = jax.lax.broadcasted_iota(jnp.int32, (S, D), 1)

    ak = jnp.abs(k)
    kmax = jnp.max(ak, axis=1, keepdims=True)
    a_idx = jnp.min(jnp.where(ak == kmax, iota, D), axis=1, keepdims=True)
    pt = (iota == a_idx).astype(jnp.float32)

    hi = jax.lax.Precision.HIGHEST
    dn = (((1,), (1,)), ((), ()))
    dc = (((1,), (0,)), ((), ()))
    A = jax.lax.dot_general(q, k, dn, preferred_element_type=jnp.float32, precision=hi)
    asum = jax.lax.dot_general(A, pt, dc, preferred_element_type=jnp.float32, precision=hi)

    work = jnp.abs(q)
    scores = []
    onehots = []
    for _ in range(_TOP_K):
        m = jnp.max(work, axis=1, keepdims=True)
        idx = jnp.min(jnp.where(work == m, iota, D), axis=1, keepdims=True)
        oh = iota == idx
        scores.append(m)
        onehots.append(oh)
        work = jnp.where(oh, -jnp.inf, work)
    es = [jnp.exp(s - scores[0]) for s in scores]
    w_norm = es[0]
    for e in es[1:]:
        w_norm = w_norm + e

    g = jnp.zeros((S, D), jnp.float32)
    for e, oh in zip(es, onehots):
        seg = jnp.sum(jnp.where(oh, asum, 0.0), axis=1, keepdims=True) + _EPS
        g = g + jnp.where(oh, (e / w_norm) / seg, 0.0)

    G = jax.lax.dot_general(g, pt, dn, preferred_element_type=jnp.float32, precision=hi)
    C = A * G
    o_ref[0, 0] = jax.lax.dot_general(C, v, dc, preferred_element_type=jnp.float32, precision=hi)


@jax.jit
def kernel(keys, values, queries):
    Bb, H, S, D = keys.shape
    spec = pl.BlockSpec((1, 1, S, D), lambda b, h: (b, h, 0, 0))
    return pl.pallas_call(
        _obm_kernel,
        grid=(Bb, H),
        in_specs=[spec, spec, spec],
        out_specs=spec,
        out_shape=jax.ShapeDtypeStruct((Bb, H, S, D), jnp.float32),
    )(keys, values, queries)
